# baseline (speedup 1.0000x reference)
import math

import jax
import jax.numpy as jnp
from jax.experimental import pallas as pl

EPS = 1e-5
SLOPE = 0.01
KNN_K = 20
KS = 24
NB = 8
NEG = -3.0e38


def _leaky(a):
    return jnp.where(a >= 0, a, SLOPE * a)


def _pd(X, E):
    C = X.shape[0]
    f32 = jnp.float32
    G = jnp.dot(X, X.T, preferred_element_type=f32)
    ncol = jnp.sum(X * X, axis=1, keepdims=True)
    nrow = jnp.sum(jnp.broadcast_to(ncol, (C, C)) * E, axis=0, keepdims=True)
    return 2.0 * G - ncol - nrow


def _gather_conv(X, S3, WT, s, b, penK):
    C, t = X.shape
    f32 = jnp.float32
    S2 = S3.reshape(C * KS, C)
    bf16 = jnp.bfloat16
    S2b = S2.astype(bf16)
    Xh = X.astype(bf16)
    Xm_f = X - Xh.astype(f32)
    Xm = Xm_f.astype(bf16)
    Xt = (Xm_f - Xm.astype(f32)).astype(bf16)
    feat = (jnp.dot(S2b, Xh, preferred_element_type=f32)
            + jnp.dot(S2b, Xm, preferred_element_type=f32)
            + jnp.dot(S2b, Xt, preferred_element_type=f32))
    fm = (feat.reshape(C, KS, t) - X[:, None, :]).reshape(C * KS, t)
    xr = jnp.broadcast_to(X[:, None, :], (C, KS, t)).reshape(C * KS, t)
    f2 = jnp.concatenate([fm, xr], axis=1)
    h3 = jnp.dot(f2, WT, preferred_element_type=f32).reshape(C, KS, -1)
    sgn = jnp.where(s >= 0, 1.0, -1.0)
    hsel = sgn * jnp.max(sgn[None, :, :] * h3 + penK, axis=1)
    return _leaky(hsel * s + b)


def _edge_block_group(Xs, WT, s, b, E, penK, tieG):
    NB = len(Xs)
    C = Xs[0].shape[0]
    f32 = jnp.float32
    pds = [_pd(X, E) for X in Xs]
    pdg = jnp.concatenate(pds, axis=1)
    Pg = jnp.broadcast_to(pdg[:, None, :], (C, C, NB * C))
    Ag = jnp.concatenate(
        [jnp.broadcast_to(
            jnp.sum(jnp.broadcast_to(pd[:, None, :], (C, C, C))
                    * E[None, :, :], axis=2, keepdims=True), (C, C, C))
         for pd in pds], axis=2)
    beats = (Ag > Pg) | ((Ag == Pg) & tieG)
    rank = jnp.sum(jnp.where(beats, 1.0, 0.0), axis=1, keepdims=True)
    kk = jax.lax.broadcasted_iota(jnp.int32, (C, KS, NB * C), 1).astype(f32)
    S3g = jnp.where((rank == kk) & (kk < float(KNN_K)), 1.0, 0.0)
    return [_gather_conv(X, S3g[:, :, n * C:(n + 1) * C], WT, s, b, penK)
            for n, X in enumerate(Xs)]


def _dgcnn_body(x_ref, W0T, s0, b0, W1T, s1, b1,
                W2T, s2, b2, W3T, s3, b3,
                WgT, sg, bg, E_ref, penK_ref, out_ref):
    cur = [x_ref[n] for n in range(NB)]
    E = E_ref[...]
    penK = penK_ref[...]
    C = cur[0].shape[0]
    ii = jax.lax.broadcasted_iota(jnp.int32, (C, C, NB * C), 1)
    jj = jax.lax.broadcasted_iota(jnp.int32, (C, C, NB * C), 2)
    tieG = ii < (jj & (C - 1))
    outs = []
    for WT, s, b in ((W0T, s0, b0), (W1T, s1, b1),
                     (W2T, s2, b2), (W3T, s3, b3)):
        cur = _edge_block_group(cur, WT[...], s[...], b[...], E, penK, tieG)
        outs.append(cur)
    for row in range(NB):
        z = jnp.concatenate([o[row] for o in outs], axis=1)
        zg = jnp.dot(z, WgT[...], preferred_element_type=jnp.float32)
        a = _leaky(zg * sg[...] + bg[...])
        out_ref[row, :, 0:1024] = jnp.max(a, axis=0, keepdims=True)
        out_ref[row, :, 1024:2048] = jnp.mean(a, axis=0, keepdims=True)


def _head_body(h_ref, Wm1T, sm1, bm1, Wm2T, sm2, bm2, WfT, bf, out_ref):
    H = h_ref[...]
    h1 = _leaky(jnp.dot(H, Wm1T[...], preferred_element_type=jnp.float32)
                * sm1[...] + bm1[...])
    h2 = _leaky(jnp.dot(h1, Wm2T[...], preferred_element_type=jnp.float32)
                * sm2[...] + bm2[...])
    out_ref[...] = jnp.dot(h2, WfT[...],
                           preferred_element_type=jnp.float32) + bf[...]


def kernel(x, W0, g0, b0, W1, g1, b1, W2, g2, b2, W3, g3, b3,
           Wg, gg, bg, Wm1, gm1, bm1, Wm2, gm2, bm2, Wf, bf):
    B, C, T = x.shape
    inv = 1.0 / math.sqrt(1.0 + EPS)

    def sb(g, bb):
        return (g * inv).reshape(1, -1), bb.reshape(1, -1)

    s0, b0v = sb(g0, b0)
    s1, b1v = sb(g1, b1)
    s2, b2v = sb(g2, b2)
    s3, b3v = sb(g3, b3)
    sg, bgv = sb(gg, bg)
    sm1, bm1v = sb(gm1, bm1)
    sm2, bm2v = sb(gm2, bm2)
    E = jnp.eye(C, dtype=jnp.float32)
    penK = jnp.where(jnp.arange(KS) < KNN_K, 0.0, NEG).astype(
        jnp.float32).reshape(1, KS, 1)

    def full(a):
        return pl.BlockSpec(a.shape, lambda i: (0,) * a.ndim)

    wargs = (W0.T, s0, b0v, W1.T, s1, b1v, W2.T, s2, b2v,
             W3.T, s3, b3v, Wg.T, sg, bgv, E, penK)
    pooled = pl.pallas_call(
        _dgcnn_body,
        grid=(B // NB,),
        in_specs=[pl.BlockSpec((NB, C, T), lambda i: (i, 0, 0))]
        + [full(a) for a in wargs],
        out_specs=pl.BlockSpec((NB, 1, 2048), lambda i: (i, 0, 0)),
        out_shape=jax.ShapeDtypeStruct((B, 1, 2048), jnp.float32),
    )(x, *wargs)
    pooled = pooled.reshape(B, 2048)

    bfv = bf.reshape(1, -1)
    hargs = (Wm1.T, sm1, bm1v, Wm2.T, sm2, bm2v, Wf.T, bfv)
    out = pl.pallas_call(
        _head_body,
        in_specs=[pl.BlockSpec(pooled.shape, lambda: (0, 0))]
        + [pl.BlockSpec(a.shape, lambda: (0, 0)) for a in hargs],
        out_specs=pl.BlockSpec((B, 4), lambda: (0, 0)),
        out_shape=jax.ShapeDtypeStruct((B, 4), jnp.float32),
    )(pooled, *hargs)
    return out


# speedup vs baseline: 1.1284x; 1.1284x over previous
import math

import jax
import jax.numpy as jnp
from jax.experimental import pallas as pl

EPS = 1e-5
SLOPE = 0.01
KNN_K = 20
KS = 24
NB = 4
NEG = -3.0e38


def _leaky(a):
    return jnp.where(a >= 0, a, SLOPE * a)


def _pd(X, E):
    C = X.shape[0]
    f32 = jnp.float32
    G = jnp.dot(X, X.T, preferred_element_type=f32)
    ncol = jnp.sum(X * X, axis=1, keepdims=True)
    nrow = jnp.sum(jnp.broadcast_to(ncol, (C, C)) * E, axis=0, keepdims=True)
    return 2.0 * G - ncol - nrow


def _gather_conv(X, S3, WT, s, b, penK):
    C, t = X.shape
    f32 = jnp.float32
    S2 = S3.reshape(C * KS, C)
    bf16 = jnp.bfloat16
    S2b = S2.astype(bf16)
    Xh = X.astype(bf16)
    Xm_f = X - Xh.astype(f32)
    Xm = Xm_f.astype(bf16)
    Xt = (Xm_f - Xm.astype(f32)).astype(bf16)
    feat = (jnp.dot(S2b, Xh, preferred_element_type=f32)
            + jnp.dot(S2b, Xm, preferred_element_type=f32)
            + jnp.dot(S2b, Xt, preferred_element_type=f32))
    fm = (feat.reshape(C, KS, t) - X[:, None, :]).reshape(C * KS, t)
    xr = jnp.broadcast_to(X[:, None, :], (C, KS, t)).reshape(C * KS, t)
    f2 = jnp.concatenate([fm, xr], axis=1)
    h3 = jnp.dot(f2, WT, preferred_element_type=f32).reshape(C, KS, -1)
    sgn = jnp.where(s >= 0, 1.0, -1.0)
    hsel = sgn * jnp.max(sgn[None, :, :] * h3 + penK, axis=1)
    return _leaky(hsel * s + b)


def _edge_block_group(Xs, WT, s, b, E, penK, tieG):
    NB = len(Xs)
    C = Xs[0].shape[0]
    f32 = jnp.float32
    pds = [_pd(X, E) for X in Xs]
    pdg = jnp.concatenate(pds, axis=1)
    Pg = jnp.broadcast_to(pdg[:, None, :], (C, C, NB * C))
    Ag = jnp.concatenate(
        [jnp.broadcast_to(
            jnp.sum(jnp.broadcast_to(pd[:, None, :], (C, C, C))
                    * E[None, :, :], axis=2, keepdims=True), (C, C, C))
         for pd in pds], axis=2)
    beats = (Ag > Pg) | ((Ag == Pg) & tieG)
    rank = jnp.sum(jnp.where(beats, 1.0, 0.0), axis=1, keepdims=True)
    kk = jax.lax.broadcasted_iota(jnp.int32, (C, KS, NB * C), 1).astype(f32)
    S3g = jnp.where((rank == kk) & (kk < float(KNN_K)), 1.0, 0.0)
    return [_gather_conv(X, S3g[:, :, n * C:(n + 1) * C], WT, s, b, penK)
            for n, X in enumerate(Xs)]


def _dgcnn_body(x_ref, W0T, s0, b0, W1T, s1, b1,
                W2T, s2, b2, W3T, s3, b3,
                WgT, sg, bg, E_ref, penK_ref, out_ref):
    cur = [x_ref[n] for n in range(NB)]
    E = E_ref[...]
    penK = penK_ref[...]
    C = cur[0].shape[0]
    ii = jax.lax.broadcasted_iota(jnp.int32, (C, C, NB * C), 1)
    jj = jax.lax.broadcasted_iota(jnp.int32, (C, C, NB * C), 2)
    tieG = ii < (jj & (C - 1))
    outs = []
    for WT, s, b in ((W0T, s0, b0), (W1T, s1, b1),
                     (W2T, s2, b2), (W3T, s3, b3)):
        cur = _edge_block_group(cur, WT[...], s[...], b[...], E, penK, tieG)
        outs.append(cur)
    for row in range(NB):
        z = jnp.concatenate([o[row] for o in outs], axis=1)
        zg = jnp.dot(z, WgT[...], preferred_element_type=jnp.float32)
        a = _leaky(zg * sg[...] + bg[...])
        out_ref[row, :, 0:1024] = jnp.max(a, axis=0, keepdims=True)
        out_ref[row, :, 1024:2048] = jnp.mean(a, axis=0, keepdims=True)


def _head_body(h_ref, Wm1T, sm1, bm1, Wm2T, sm2, bm2, WfT, bf, out_ref):
    H = h_ref[...]
    h1 = _leaky(jnp.dot(H, Wm1T[...], preferred_element_type=jnp.float32)
                * sm1[...] + bm1[...])
    h2 = _leaky(jnp.dot(h1, Wm2T[...], preferred_element_type=jnp.float32)
                * sm2[...] + bm2[...])
    out_ref[...] = jnp.dot(h2, WfT[...],
                           preferred_element_type=jnp.float32) + bf[...]


def kernel(x, W0, g0, b0, W1, g1, b1, W2, g2, b2, W3, g3, b3,
           Wg, gg, bg, Wm1, gm1, bm1, Wm2, gm2, bm2, Wf, bf):
    B, C, T = x.shape
    inv = 1.0 / math.sqrt(1.0 + EPS)

    def sb(g, bb):
        return (g * inv).reshape(1, -1), bb.reshape(1, -1)

    s0, b0v = sb(g0, b0)
    s1, b1v = sb(g1, b1)
    s2, b2v = sb(g2, b2)
    s3, b3v = sb(g3, b3)
    sg, bgv = sb(gg, bg)
    sm1, bm1v = sb(gm1, bm1)
    sm2, bm2v = sb(gm2, bm2)
    E = jnp.eye(C, dtype=jnp.float32)
    penK = jnp.where(jnp.arange(KS) < KNN_K, 0.0, NEG).astype(
        jnp.float32).reshape(1, KS, 1)

    def full(a):
        return pl.BlockSpec(a.shape, lambda i: (0,) * a.ndim)

    wargs = (W0.T, s0, b0v, W1.T, s1, b1v, W2.T, s2, b2v,
             W3.T, s3, b3v, Wg.T, sg, bgv, E, penK)
    pooled = pl.pallas_call(
        _dgcnn_body,
        grid=(B // NB,),
        in_specs=[pl.BlockSpec((NB, C, T), lambda i: (i, 0, 0))]
        + [full(a) for a in wargs],
        out_specs=pl.BlockSpec((NB, 1, 2048), lambda i: (i, 0, 0)),
        out_shape=jax.ShapeDtypeStruct((B, 1, 2048), jnp.float32),
    )(x, *wargs)
    pooled = pooled.reshape(B, 2048)

    bfv = bf.reshape(1, -1)
    hargs = (Wm1.T, sm1, bm1v, Wm2.T, sm2, bm2v, Wf.T, bfv)
    out = pl.pallas_call(
        _head_body,
        in_specs=[pl.BlockSpec(pooled.shape, lambda: (0, 0))]
        + [pl.BlockSpec(a.shape, lambda: (0, 0)) for a in hargs],
        out_specs=pl.BlockSpec((B, 4), lambda: (0, 0)),
        out_shape=jax.ShapeDtypeStruct((B, 4), jnp.float32),
    )(pooled, *hargs)
    return out
